# baseline (speedup 1.0000x reference)
import dataclasses
import functools

import jax
import jax.numpy as jnp
from jax import lax
from jax.experimental import pallas as pl
from jax.experimental.pallas import tpu as pltpu
from jax.experimental.pallas import tpu_sc as plsc

_N, _E, _D, _DE, _L = 10000, 320000, 128, 16, 3
_CH = 128
_BE = 8000
_NSUB = 16
_RPS_MAIN = 632
_RPS_LAST = _N - 15 * _RPS_MAIN

_vec_mesh = plsc.VectorSubcoreMesh(core_axis_name="core",
                                   subcore_axis_name="subcore")

_sc_params = pltpu.CompilerParams()
if "needs_layout_passes" in pltpu.CompilerParams.__dataclass_fields__:
    _sc_params = dataclasses.replace(_sc_params, needs_layout_passes=False)


def _sc_message(h, e, src2d, dst2d, ewq, zeros):
    @functools.partial(
        pl.kernel,
        out_type=jax.ShapeDtypeStruct((2, _N, _D), jnp.float32),
        mesh=_vec_mesh,
        compiler_params=_sc_params,
        scratch_types=[pltpu.VMEM_SHARED((_N, _D), jnp.float32),
                       pltpu.VMEM((_CH, _D), jnp.float32),
                       pltpu.SemaphoreType.DMA],
    )
    def k(h_hbm, e_hbm, si_hbm, di_hbm, w_hbm, z_hbm, o_hbm,
          acc, g_vmem, sem):
        cid = lax.axis_index("core")
        sid = lax.axis_index("subcore")
        r0 = sid * _RPS_MAIN

        @pl.when(sid < _NSUB - 1)
        def _():
            pltpu.sync_copy(z_hbm.at[pl.ds(r0, _RPS_MAIN)],
                            acc.at[pl.ds(r0, _RPS_MAIN)])

        @pl.when(sid == _NSUB - 1)
        def _():
            pltpu.sync_copy(z_hbm.at[pl.ds(r0, _RPS_LAST)],
                            acc.at[pl.ds(r0, _RPS_LAST)])

        plsc.subcore_barrier()

        zero16 = jnp.zeros((16,), jnp.int32)

        def body(si_vmem, e_vmem, w_vmem, di_vmem):
            pltpu.async_copy(h_hbm.at[si_vmem.at[0]], g_vmem, sem).wait()

            @pl.loop(0, _CH)
            def _(j):
                wj = plsc.load_gather(
                    w_vmem, [zero16, jnp.full((16,), j, jnp.int32)])
                for v in range(8):
                    sl = pl.ds(v * 16, 16)
                    g_vmem[j, sl] = jnp.maximum(
                        g_vmem[j, sl] + e_vmem[j, sl], 0.0) * wj

            pltpu.sync_copy(g_vmem, acc.at[di_vmem.at[0]], add=True)

        pltpu.emit_pipeline(
            body,
            grid=(_E // _CH,),
            in_specs=[pl.BlockSpec((1, _CH), lambda i: (0, i)),
                      pl.BlockSpec((_CH, _D), lambda i: (i, 0)),
                      pl.BlockSpec((1, _CH), lambda i: (i, 0)),
                      pl.BlockSpec((1, _CH), lambda i: (0, i))],
            out_specs=[],
            core_axis_name=("core", "subcore"),
            dimension_semantics=(pltpu.PARALLEL,),
        )(si_hbm, e_hbm, w_hbm, di_hbm)

        plsc.subcore_barrier()

        @pl.when(sid < _NSUB - 1)
        def _():
            pltpu.sync_copy(acc.at[pl.ds(r0, _RPS_MAIN)],
                            o_hbm.at[cid, pl.ds(r0, _RPS_MAIN)])

        @pl.when(sid == _NSUB - 1)
        def _():
            pltpu.sync_copy(acc.at[pl.ds(r0, _RPS_LAST)],
                            o_hbm.at[cid, pl.ds(r0, _RPS_LAST)])

    return k(h, e, src2d, dst2d, ewq, zeros)


def _tc_bond(ea, wb1, bb1, wb2, bb2):
    def body(ea_ref, w1_ref, b1_ref, w2_ref, b2_ref, e_ref):
        t = jnp.maximum(
            jnp.dot(ea_ref[...], w1_ref[...],
                    preferred_element_type=jnp.float32) + b1_ref[...], 0.0)
        e_ref[...] = jnp.dot(t, w2_ref[...],
                             preferred_element_type=jnp.float32) + b2_ref[...]

    return pl.pallas_call(
        body,
        grid=(_E // _BE,),
        in_specs=[pl.BlockSpec((_BE, _DE), lambda i: (i, 0)),
                  pl.BlockSpec((_DE, _D), lambda i: (0, 0)),
                  pl.BlockSpec((1, _D), lambda i: (0, 0)),
                  pl.BlockSpec((_D, _D), lambda i: (0, 0)),
                  pl.BlockSpec((1, _D), lambda i: (0, 0))],
        out_specs=pl.BlockSpec((_BE, _D), lambda i: (i, 0)),
        out_shape=jax.ShapeDtypeStruct((_E, _D), jnp.float32),
    )(ea, wb1, bb1, wb2, bb2)


def _node_update(h, p, ope, wm1, bm1, wm2, bm2, gam, bet):
    z = h * ope + p[0] + p[1]
    y = jnp.maximum(
        jnp.dot(z, wm1, preferred_element_type=jnp.float32) + bm1, 0.0)
    y = jnp.dot(y, wm2, preferred_element_type=jnp.float32) + bm2
    mu = jnp.mean(y, axis=0, keepdims=True)
    var = jnp.mean(jnp.square(y - mu), axis=0, keepdims=True)
    yn = (y - mu) * lax.rsqrt(var + 1e-5) * gam + bet
    return jnp.maximum(yn, 0.0)


def _tc_node(h, parts, ope, wm1, bm1, wm2, bm2, gam, bet):
    def body(h_ref, p_ref, ope_ref, w1_ref, b1_ref, w2_ref, b2_ref,
             g_ref, be_ref, o_ref):
        o_ref[...] = _node_update(h_ref[...], p_ref, ope_ref[...],
                                  w1_ref[...], b1_ref[...], w2_ref[...],
                                  b2_ref[...], g_ref[...], be_ref[...])

    return pl.pallas_call(
        body,
        out_shape=jax.ShapeDtypeStruct((_N, _D), jnp.float32),
    )(h, parts, ope, wm1, bm1, wm2, bm2, gam, bet)


def _tc_node_final(h, parts, ope, wm1, bm1, wm2, bm2, gam, bet,
                   w1, b1, w2, b2, w3, b3, w4, b4):
    def body(h_ref, p_ref, ope_ref, wm1_ref, bm1_ref, wm2_ref, bm2_ref,
             g_ref, be_ref, w1_ref, b1_ref, w2_ref, b2_ref, w3_ref, b3_ref,
             w4_ref, b4_ref, o_ref):
        hn = _node_update(h_ref[...], p_ref, ope_ref[...],
                          wm1_ref[...], bm1_ref[...], wm2_ref[...],
                          bm2_ref[...], g_ref[...], be_ref[...])
        gv = jnp.mean(hn, axis=0, keepdims=True)
        gv = jnp.maximum(jnp.dot(gv, w1_ref[...],
                                 preferred_element_type=jnp.float32)
                         + b1_ref[...], 0.0)
        gv = jnp.maximum(jnp.dot(gv, w2_ref[...],
                                 preferred_element_type=jnp.float32)
                         + b2_ref[...], 0.0)
        gv = jnp.maximum(jnp.dot(gv, w3_ref[...],
                                 preferred_element_type=jnp.float32)
                         + b3_ref[...], 0.0)
        o_ref[...] = jnp.dot(gv, w4_ref[...],
                             preferred_element_type=jnp.float32) + b4_ref[...]

    return pl.pallas_call(
        body,
        out_shape=jax.ShapeDtypeStruct((1, 1), jnp.float32),
    )(h, parts, ope, wm1, bm1, wm2, bm2, gam, bet,
      w1, b1, w2, b2, w3, b3, w4, b4)


def kernel(x, edge_index, edge_attr, edge_weight, Wb1, bb1, Wb2, bb2,
           Wm1, bm1, Wm2, bm2, eps, gamma, beta,
           W1, b1, W2, b2, W3, b3, W4, b4):
    src2d = edge_index[0].reshape(1, _E)
    dst2d = edge_index[1].reshape(1, _E)
    ewq = edge_weight.reshape(_E // _CH, _CH)
    zeros = jnp.zeros((_N, _D), jnp.float32)

    es = [_tc_bond(edge_attr, Wb1[l], bb1[l].reshape(1, _D),
                   Wb2[l], bb2[l].reshape(1, _D)) for l in range(_L)]

    h = x
    out = None
    for l in range(_L):
        parts = _sc_message(h, es[l], src2d, dst2d, ewq, zeros)
        ope = (1.0 + eps[l]).reshape(1, 1)
        args = (h, parts, ope,
                Wm1[l], bm1[l].reshape(1, _D),
                Wm2[l], bm2[l].reshape(1, _D),
                gamma[l].reshape(1, _D), beta[l].reshape(1, _D))
        if l < _L - 1:
            h = _tc_node(*args)
        else:
            out = _tc_node_final(*args,
                                 W1, b1.reshape(1, _D),
                                 W2, b2.reshape(1, _D),
                                 W3, b3.reshape(1, _D),
                                 W4, b4.reshape(1, 1))
    return out


# speedup vs baseline: 1.9381x; 1.9381x over previous
import dataclasses
import functools

import jax
import jax.numpy as jnp
from jax import lax
from jax.experimental import pallas as pl
from jax.experimental.pallas import tpu as pltpu
from jax.experimental.pallas import tpu_sc as plsc

_N, _E, _D, _DE, _L = 10000, 320000, 128, 16, 3
_CH = 128
_BE = 8000
_NSUB = 16
_RPS_MAIN = 632
_RPS_LAST = _N - 15 * _RPS_MAIN

_vec_mesh = plsc.VectorSubcoreMesh(core_axis_name="core",
                                   subcore_axis_name="subcore")

_sc_params = pltpu.CompilerParams()
if "needs_layout_passes" in pltpu.CompilerParams.__dataclass_fields__:
    _sc_params = dataclasses.replace(_sc_params, needs_layout_passes=False)


def _sc_message(h, e, src2d, dst2d, ewq, zeros):
    @functools.partial(
        pl.kernel,
        out_type=jax.ShapeDtypeStruct((2, _N, _D), jnp.float32),
        mesh=_vec_mesh,
        compiler_params=_sc_params,
        scratch_types=[pltpu.VMEM_SHARED((_N, _D), jnp.float32),
                       pltpu.VMEM((_CH, _D), jnp.float32),
                       pltpu.SemaphoreType.DMA],
    )
    def k(h_hbm, e_hbm, si_hbm, di_hbm, w_hbm, z_hbm, o_hbm,
          acc, g_vmem, sem):
        cid = lax.axis_index("core")
        sid = lax.axis_index("subcore")
        r0 = sid * _RPS_MAIN

        @pl.when(sid < _NSUB - 1)
        def _():
            pltpu.sync_copy(z_hbm.at[pl.ds(r0, _RPS_MAIN)],
                            acc.at[pl.ds(r0, _RPS_MAIN)])

        @pl.when(sid == _NSUB - 1)
        def _():
            pltpu.sync_copy(z_hbm.at[pl.ds(r0, _RPS_LAST)],
                            acc.at[pl.ds(r0, _RPS_LAST)])

        plsc.subcore_barrier()

        zero16 = jnp.zeros((16,), jnp.int32)

        def body(si_vmem, e_vmem, w_vmem, di_vmem):
            pltpu.async_copy(h_hbm.at[si_vmem.at[0]], g_vmem, sem).wait()

            @plsc.parallel_loop(0, _CH, unroll=4)
            def _(j):
                wj = plsc.load_gather(
                    w_vmem, [zero16, jnp.full((16,), j, jnp.int32)])
                for v in range(8):
                    sl = pl.ds(v * 16, 16)
                    g_vmem[j, sl] = jnp.maximum(
                        g_vmem[j, sl] + e_vmem[j, sl], 0.0) * wj

            pltpu.sync_copy(g_vmem, acc.at[di_vmem.at[0]], add=True)

        pltpu.emit_pipeline(
            body,
            grid=(_E // _CH,),
            in_specs=[pl.BlockSpec((1, _CH), lambda i: (0, i)),
                      pl.BlockSpec((_CH, _D), lambda i: (i, 0)),
                      pl.BlockSpec((1, _CH), lambda i: (i, 0)),
                      pl.BlockSpec((1, _CH), lambda i: (0, i))],
            out_specs=[],
            core_axis_name=("core", "subcore"),
            dimension_semantics=(pltpu.PARALLEL,),
        )(si_hbm, e_hbm, w_hbm, di_hbm)

        plsc.subcore_barrier()

        @pl.when(sid < _NSUB - 1)
        def _():
            pltpu.sync_copy(acc.at[pl.ds(r0, _RPS_MAIN)],
                            o_hbm.at[cid, pl.ds(r0, _RPS_MAIN)])

        @pl.when(sid == _NSUB - 1)
        def _():
            pltpu.sync_copy(acc.at[pl.ds(r0, _RPS_LAST)],
                            o_hbm.at[cid, pl.ds(r0, _RPS_LAST)])

    return k(h, e, src2d, dst2d, ewq, zeros)


def _tc_bond(ea, wb1, bb1, wb2, bb2):
    def body(ea_ref, w1_ref, b1_ref, w2_ref, b2_ref, e_ref):
        t = jnp.maximum(
            jnp.dot(ea_ref[...], w1_ref[...],
                    preferred_element_type=jnp.float32) + b1_ref[...], 0.0)
        e_ref[...] = jnp.dot(t, w2_ref[...],
                             preferred_element_type=jnp.float32) + b2_ref[...]

    return pl.pallas_call(
        body,
        grid=(_E // _BE,),
        in_specs=[pl.BlockSpec((_BE, _DE), lambda i: (i, 0)),
                  pl.BlockSpec((_DE, _D), lambda i: (0, 0)),
                  pl.BlockSpec((1, _D), lambda i: (0, 0)),
                  pl.BlockSpec((_D, _D), lambda i: (0, 0)),
                  pl.BlockSpec((1, _D), lambda i: (0, 0))],
        out_specs=pl.BlockSpec((_BE, _D), lambda i: (i, 0)),
        out_shape=jax.ShapeDtypeStruct((_E, _D), jnp.float32),
    )(ea, wb1, bb1, wb2, bb2)


def _node_update(h, p, ope, wm1, bm1, wm2, bm2, gam, bet):
    z = h * ope + p[0] + p[1]
    y = jnp.maximum(
        jnp.dot(z, wm1, preferred_element_type=jnp.float32) + bm1, 0.0)
    y = jnp.dot(y, wm2, preferred_element_type=jnp.float32) + bm2
    mu = jnp.mean(y, axis=0, keepdims=True)
    var = jnp.mean(jnp.square(y - mu), axis=0, keepdims=True)
    yn = (y - mu) * lax.rsqrt(var + 1e-5) * gam + bet
    return jnp.maximum(yn, 0.0)


def _tc_node(h, parts, ope, wm1, bm1, wm2, bm2, gam, bet):
    def body(h_ref, p_ref, ope_ref, w1_ref, b1_ref, w2_ref, b2_ref,
             g_ref, be_ref, o_ref):
        o_ref[...] = _node_update(h_ref[...], p_ref, ope_ref[...],
                                  w1_ref[...], b1_ref[...], w2_ref[...],
                                  b2_ref[...], g_ref[...], be_ref[...])

    return pl.pallas_call(
        body,
        out_shape=jax.ShapeDtypeStruct((_N, _D), jnp.float32),
    )(h, parts, ope, wm1, bm1, wm2, bm2, gam, bet)


def _tc_node_final(h, parts, ope, wm1, bm1, wm2, bm2, gam, bet,
                   w1, b1, w2, b2, w3, b3, w4, b4):
    def body(h_ref, p_ref, ope_ref, wm1_ref, bm1_ref, wm2_ref, bm2_ref,
             g_ref, be_ref, w1_ref, b1_ref, w2_ref, b2_ref, w3_ref, b3_ref,
             w4_ref, b4_ref, o_ref):
        hn = _node_update(h_ref[...], p_ref, ope_ref[...],
                          wm1_ref[...], bm1_ref[...], wm2_ref[...],
                          bm2_ref[...], g_ref[...], be_ref[...])
        gv = jnp.mean(hn, axis=0, keepdims=True)
        gv = jnp.maximum(jnp.dot(gv, w1_ref[...],
                                 preferred_element_type=jnp.float32)
                         + b1_ref[...], 0.0)
        gv = jnp.maximum(jnp.dot(gv, w2_ref[...],
                                 preferred_element_type=jnp.float32)
                         + b2_ref[...], 0.0)
        gv = jnp.maximum(jnp.dot(gv, w3_ref[...],
                                 preferred_element_type=jnp.float32)
                         + b3_ref[...], 0.0)
        o_ref[...] = jnp.dot(gv, w4_ref[...],
                             preferred_element_type=jnp.float32) + b4_ref[...]

    return pl.pallas_call(
        body,
        out_shape=jax.ShapeDtypeStruct((1, 1), jnp.float32),
    )(h, parts, ope, wm1, bm1, wm2, bm2, gam, bet,
      w1, b1, w2, b2, w3, b3, w4, b4)


def kernel(x, edge_index, edge_attr, edge_weight, Wb1, bb1, Wb2, bb2,
           Wm1, bm1, Wm2, bm2, eps, gamma, beta,
           W1, b1, W2, b2, W3, b3, W4, b4):
    src2d = edge_index[0].reshape(1, _E)
    dst2d = edge_index[1].reshape(1, _E)
    ewq = edge_weight.reshape(_E // _CH, _CH)
    zeros = jnp.zeros((_N, _D), jnp.float32)

    es = [_tc_bond(edge_attr, Wb1[l], bb1[l].reshape(1, _D),
                   Wb2[l], bb2[l].reshape(1, _D)) for l in range(_L)]

    h = x
    out = None
    for l in range(_L):
        parts = _sc_message(h, es[l], src2d, dst2d, ewq, zeros)
        ope = (1.0 + eps[l]).reshape(1, 1)
        args = (h, parts, ope,
                Wm1[l], bm1[l].reshape(1, _D),
                Wm2[l], bm2[l].reshape(1, _D),
                gamma[l].reshape(1, _D), beta[l].reshape(1, _D))
        if l < _L - 1:
            h = _tc_node(*args)
        else:
            out = _tc_node_final(*args,
                                 W1, b1.reshape(1, _D),
                                 W2, b2.reshape(1, _D),
                                 W3, b3.reshape(1, _D),
                                 W4, b4.reshape(1, 1))
    return out
